# baseline (speedup 1.0000x reference)
import functools

import jax
import jax.numpy as jnp
from jax import lax
from jax.experimental import pallas as pl
from jax.experimental.pallas import tpu as pltpu
from jax.experimental.pallas import tpu_sc as plsc

NUM_CORES = 2
NUM_SUBCORES = 16
NW = NUM_CORES * NUM_SUBCORES
L = 16

BATCH = 16384
D = 64
NUM_ROWS = 1000000
RANGE = 32768
CW = 256
FULL_CHUNKS = RANGE // CW
W30_REG = (999936 - 30 * RANGE) // CW
TAIL_U0 = 999936
DUMMY = BATCH
CAP = 20480
TOTROWS = NW * CAP


def _sweep_body(users_hbm, items_hbm, utabT, itabT, tailTu, tailTi,
                rows_u, rows_i, pb_u, pb_i, cnt_u, cnt_i,
                idxbuf, mu, mb, cu, cb, buf, tbuf, obuf, oent, cbuf, sem):
    w = lax.axis_index("s") * NUM_CORES + lax.axis_index("c")
    base_w = w * RANGE
    reg_chunks = jnp.where(w == 30, W30_REG,
                           jnp.where(w == 31, 0, FULL_CHUNKS))
    iota = lax.iota(jnp.int32, L)

    for idx_hbm, tabT, tailT, rows_out, pb_out, cnt_out in (
            (users_hbm, utabT, tailTu, rows_u, pb_u, cnt_u),
            (items_hbm, itabT, tailTi, rows_i, pb_i, cnt_i)):
        pltpu.sync_copy(idx_hbm, idxbuf)

        def scan(k, cnt):
            b0 = pl.multiple_of(k * L, L)
            u_vec = idxbuf[pl.ds(b0, L)]
            m = (u_vec >> 15) == w
            plsc.store_compressed(mu.at[pl.ds(cnt, L)], u_vec, mask=m)
            plsc.store_compressed(mb.at[pl.ds(cnt, L)], b0 + iota, mask=m)
            return cnt + plsc.all_reduce_population_count(m)[0]

        mcnt = lax.fori_loop(0, BATCH // L, scan, 0)
        mticks = (mcnt + L - 1) // L

        def chunk_step(c, ocnt):
            is_reg = c < reg_chunks
            is_tail = (w == 30) & (c == W30_REG)
            u0 = pl.multiple_of(base_w + c * CW, 128)

            @pl.when(is_reg)
            def _():
                cps = [pltpu.async_copy(
                    tabT.at[pl.ds(dh * 8, 8), pl.ds(u0, CW)],
                    buf.at[dh], sem) for dh in range(8)]
                for cp in cps:
                    cp.wait()

            @pl.when(is_tail)
            def _():
                cps = [pltpu.async_copy(
                    tailT.at[pl.ds(dh * 8, 8), :],
                    buf.at[dh, :, pl.ds(0, 128)], sem) for dh in range(8)]
                for cp in cps:
                    cp.wait()

            def collect(j, ccnt):
                p0 = pl.multiple_of(j * L, L)
                u_vec = mu[pl.ds(p0, L)]
                b_vec = mb[pl.ds(p0, L)]
                m = ((p0 + iota) < mcnt) & (((u_vec - base_w) >> 8) == c)
                plsc.store_compressed(cu.at[pl.ds(ccnt, L)], u_vec, mask=m)
                plsc.store_compressed(cb.at[pl.ds(ccnt, L)], b_vec, mask=m)
                return ccnt + plsc.all_reduce_population_count(m)[0]

            ccnt = lax.fori_loop(0, mticks, collect, 0)
            cticks = (ccnt + L - 1) // L

            def batch(e, ocnt_in):
                p0 = pl.multiple_of(e * L, L)
                u_vec = cu[pl.ds(p0, L)]
                b_vec = cb[pl.ds(p0, L)]
                vmask = (p0 + iota) < ccnt
                uloc = (u_vec - u0) & (CW - 1)
                for q in range(D):
                    g = plsc.load_gather(
                        buf, [jnp.full((L,), q >> 3, jnp.int32),
                              jnp.full((L,), q & 7, jnp.int32), uloc],
                        mask=vmask)
                    tbuf[q] = g
                om = pl.multiple_of(ocnt_in & 127, L)
                for l in range(L):
                    for qq in range(D // L):
                        r = plsc.load_gather(
                            tbuf, [qq * L + iota, jnp.full((L,), l, jnp.int32)])
                        obuf[om + l, pl.ds(qq * L, L)] = r
                oent[pl.ds(om, L)] = jnp.where(vmask, b_vec, DUMMY)

                @pl.when((ocnt_in & 127) == 112)
                def _():
                    g0 = pl.multiple_of(w * CAP + (ocnt_in - 112), 128)
                    pltpu.sync_copy(obuf, rows_out.at[pl.ds(g0, 128), :])
                    pltpu.sync_copy(oent, pb_out.at[pl.ds(g0, 128)])

                return ocnt_in + L

            return lax.fori_loop(0, cticks, batch, ocnt)

        ocnt = lax.fori_loop(0, FULL_CHUNKS + 1, chunk_step, 0)
        g0 = pl.multiple_of(w * CAP + (ocnt & ~jnp.int32(127)), 128)
        pltpu.sync_copy(obuf, rows_out.at[pl.ds(g0, 128), :])
        pltpu.sync_copy(oent, pb_out.at[pl.ds(g0, 128)])
        for j in range(4):
            cbuf[pl.ds(j * L, L)] = jnp.broadcast_to(ocnt, (L,))
        pltpu.sync_copy(cbuf, cnt_out.at[pl.ds(w * 64, 64)])


def _dot_body(rows_u, rows_i, pb_u, pb_i, cnt_u, cnt_i, out_hbm,
              cbuf, pbbuf, pos_u, pos_i, bu, bi, scratch, out_v, sem):
    w = lax.axis_index("s") * NUM_CORES + lax.axis_index("c")
    iota = lax.iota(jnp.int32, L)
    iota16 = iota * L

    for cnt_in, pb_in, pos in ((cnt_u, pb_u, pos_u), (cnt_i, pb_i, pos_i)):
        pltpu.sync_copy(cnt_in, cbuf)

        for v in range(NW):
            cnt_v = cbuf[pl.ds(v * 64, L)][0]
            npieces = (cnt_v + 2047) // 2048

            def piece(p, _):
                pltpu.sync_copy(
                    pb_in.at[pl.ds(pl.multiple_of(v * CAP + p * 2048, 8),
                                   2048)], pbbuf)

                def scan(j, _2):
                    b_vec = pbbuf[pl.ds(pl.multiple_of(j * L, L), L)]
                    e = p * 2048 + j * L + iota
                    m = (e < cnt_v) & ((b_vec >> 9) == w)
                    bl = b_vec & 511
                    plsc.store_scatter(pos, [bl >> 7, bl & 127],
                                       v * CAP + e, mask=m)
                    return 0

                jticks = jnp.minimum(2048 // L,
                                     (cnt_v - p * 2048 + L - 1) // L)
                lax.fori_loop(0, jticks, scan, 0)
                return 0

            lax.fori_loop(0, npieces, piece, 0)

    def sub(s, _):
        cpu = pltpu.async_copy(rows_u.at[pos_u.at[s]], bu, sem)
        cpi = pltpu.async_copy(rows_i.at[pos_i.at[s]], bi, sem)
        cpu.wait()
        cpi.wait()

        def group(g, _2):
            for k in range(L):
                r = g * L + k
                acc = bu[r, pl.ds(0, L)] * bi[r, pl.ds(0, L)]
                for c in range(1, D // L):
                    acc = acc + (bu[r, pl.ds(c * L, L)]
                                 * bi[r, pl.ds(c * L, L)])
                scratch[pl.ds(k * L, L)] = acc
            res = plsc.load_gather(scratch, [iota16])
            for j in range(1, L):
                res = res + plsc.load_gather(scratch, [iota16 + j])
            out_v[pl.ds(pl.multiple_of(s * 128 + g * L, L), L)] = res
            return 0

        lax.fori_loop(0, 8, group, 0)
        return 0

    lax.fori_loop(0, 4, sub, 0)
    pltpu.sync_copy(out_v, out_hbm.at[pl.ds(w * 512, 512)])


@jax.jit
def _bpr_sc(users, items, user_table, item_table):
    utabT = user_table.T
    itabT = item_table.T
    pad = ((0, 0), (0, 128 - (NUM_ROWS - TAIL_U0)))
    tailTu = jnp.pad(utabT[:, TAIL_U0:], pad)
    tailTi = jnp.pad(itabT[:, TAIL_U0:], pad)

    mesh = plsc.VectorSubcoreMesh(
        core_axis_name="c", subcore_axis_name="s",
        num_cores=NUM_CORES, num_subcores=NUM_SUBCORES)

    rows_u, rows_i, pb_u, pb_i, cnt_u, cnt_i = pl.kernel(
        _sweep_body,
        out_type=(jax.ShapeDtypeStruct((TOTROWS, D), jnp.float32),
                  jax.ShapeDtypeStruct((TOTROWS, D), jnp.float32),
                  jax.ShapeDtypeStruct((TOTROWS,), jnp.int32),
                  jax.ShapeDtypeStruct((TOTROWS,), jnp.int32),
                  jax.ShapeDtypeStruct((NW * 64,), jnp.int32),
                  jax.ShapeDtypeStruct((NW * 64,), jnp.int32)),
        mesh=mesh,
        compiler_params=pltpu.CompilerParams(
            needs_layout_passes=False, use_tc_tiling_on_sc=True),
        scratch_types=[
            pltpu.VMEM((BATCH,), jnp.int32),
            pltpu.VMEM((BATCH,), jnp.int32),
            pltpu.VMEM((BATCH,), jnp.int32),
            pltpu.VMEM((BATCH,), jnp.int32),
            pltpu.VMEM((BATCH,), jnp.int32),
            pltpu.VMEM((8, 8, CW), jnp.float32),
            pltpu.VMEM((D, L), jnp.float32),
            pltpu.VMEM((128, D), jnp.float32),
            pltpu.VMEM((128,), jnp.int32),
            pltpu.VMEM((64,), jnp.int32),
            pltpu.SemaphoreType.DMA,
        ],
    )(users, items, utabT, itabT, tailTu, tailTi)

    return pl.kernel(
        _dot_body,
        out_type=jax.ShapeDtypeStruct((BATCH,), jnp.float32),
        mesh=mesh,
        compiler_params=pltpu.CompilerParams(
            needs_layout_passes=False, use_tc_tiling_on_sc=False),
        scratch_types=[
            pltpu.VMEM((NW * 64,), jnp.int32),
            pltpu.VMEM((2048,), jnp.int32),
            pltpu.VMEM((4, 128), jnp.int32),
            pltpu.VMEM((4, 128), jnp.int32),
            pltpu.VMEM((128, D), jnp.float32),
            pltpu.VMEM((128, D), jnp.float32),
            pltpu.VMEM((L * L,), jnp.float32),
            pltpu.VMEM((512,), jnp.float32),
            pltpu.SemaphoreType.DMA,
        ],
    )(rows_u, rows_i, pb_u, pb_i, cnt_u, cnt_i)


def kernel(users, items, user_table, item_table):
    return _bpr_sc(users.astype(jnp.int32), items.astype(jnp.int32),
                   user_table, item_table)
